# Initial kernel scaffold; baseline (speedup 1.0000x reference)
import jax
import jax.numpy as jnp
from jax.experimental import pallas as pl


def kernel(x, psi, quad_weights, weight, bias):
    raise NotImplementedError("write your pallas kernel here")


# speedup vs baseline: 11.0900x; 11.0900x over previous
import jax
import jax.numpy as jnp
from jax.experimental import pallas as pl
from jax.experimental.pallas import tpu as pltpu

_B, _C, _F = 2, 64, 64
_NLAT_IN, _NLON_IN = 64, 128
_NLAT_OUT, _NLON_OUT = 32, 64
_K = 3
_ROWS = 6
_RPAD = 8


def _row_start(t: int) -> int:
    return min(max(2 * t - 2, 0), _NLAT_IN - _ROWS)


def _disco_kernel(psw_ref, x_ref, w_ref, b_ref, out_ref):
    t = pl.program_id(0)
    i0 = jnp.clip(2 * t - 2, 0, _NLAT_IN - _ROWS)
    rowp = jax.lax.broadcasted_iota(jnp.int32, (_NLON_OUT, _NLON_IN), 0)
    ys = []
    for k in range(_K):
        acc = None
        for r in range(_ROWS):
            v = psw_ref[0, k, r, :]
            ct = jnp.broadcast_to(v[None, :], (_NLON_OUT, _NLON_IN))
            for bit in range(6):
                rolled = pltpu.roll(ct, 2 << bit, axis=1)
                ct = jnp.where((rowp & (1 << bit)) != 0, rolled, ct)
            xt = x_ref[i0 + r]
            part = jax.lax.dot(ct, xt, preferred_element_type=jnp.float32)
            acc = part if acc is None else acc + part
        ys.append(acc)
    for b in range(_B):
        ob = None
        for k in range(_K):
            q = jax.lax.dot(ys[k][:, b * _C:(b + 1) * _C], w_ref[k],
                            preferred_element_type=jnp.float32)
            ob = q if ob is None else ob + q
        out_ref[0, b] = ob + b_ref[:, :]


def kernel(x, psi, quad_weights, weight, bias):
    xt = x.reshape(_B * _C, _NLAT_IN, _NLON_IN).transpose(1, 2, 0)
    psiR = psi.reshape(_K, _NLAT_OUT, _NLAT_IN, _NLON_IN)
    psw = jnp.stack([
        psiR[:, t, _row_start(t):_row_start(t) + _ROWS, :]
        * quad_weights[_row_start(t):_row_start(t) + _ROWS, :][None]
        for t in range(_NLAT_OUT)
    ])
    psw = jnp.pad(psw, ((0, 0), (0, 0), (0, _RPAD - _ROWS), (0, 0)))
    wt = jnp.transpose(weight, (2, 1, 0))
    br = bias.reshape(1, _F)
    out = pl.pallas_call(
        _disco_kernel,
        grid=(_NLAT_OUT,),
        in_specs=[
            pl.BlockSpec((1, _K, _RPAD, _NLON_IN), lambda t: (t, 0, 0, 0)),
            pl.BlockSpec((_NLAT_IN, _NLON_IN, _B * _C), lambda t: (0, 0, 0)),
            pl.BlockSpec((_K, _C, _F), lambda t: (0, 0, 0)),
            pl.BlockSpec((1, _F), lambda t: (0, 0)),
        ],
        out_specs=pl.BlockSpec((1, _B, _NLON_OUT, _F), lambda t: (t, 0, 0, 0)),
        out_shape=jax.ShapeDtypeStruct((_NLAT_OUT, _B, _NLON_OUT, _F),
                                       jnp.float32),
    )(psw, xt, wt, br)
    return out.transpose(1, 3, 0, 2)


# speedup vs baseline: 13.8240x; 1.2465x over previous
import jax
import jax.numpy as jnp
from jax.experimental import pallas as pl
from jax.experimental.pallas import tpu as pltpu

_B, _C, _F = 2, 64, 64
_NLAT_IN, _NLON_IN = 64, 128
_NLAT_OUT, _NLON_OUT = 32, 64
_K = 3
_ROWS = 6
_RPAD = 8


def _row_start(t: int) -> int:
    return min(max(2 * t - 2, 0), _NLAT_IN - _ROWS)


def _disco_kernel(psw_ref, x_ref, wbd_ref, b_ref, out_ref):
    t = pl.program_id(0)
    i0 = jnp.clip(2 * t - 2, 0, _NLAT_IN - _ROWS)
    xw = x_ref[pl.ds(i0, _ROWS)]
    X = xw.reshape(_ROWS * _NLON_IN, _B * _C)
    kblocks = []
    for k in range(_K):
        rs = []
        for r in range(_ROWS):
            v = psw_ref[0, k, r, :]
            ct0 = jnp.broadcast_to(v[None, :], (_NLON_OUT, _NLON_IN))
            rs.append(pltpu.roll(ct0, 0, axis=1, stride=2, stride_axis=0))
        kblocks.append(jnp.concatenate(rs, axis=1))
    ct = jnp.concatenate(kblocks, axis=0)
    y = jax.lax.dot(ct, X, preferred_element_type=jnp.float32)
    ob = None
    for k in range(_K):
        q = jax.lax.dot(y[k * _NLON_OUT:(k + 1) * _NLON_OUT, :], wbd_ref[k],
                        preferred_element_type=jnp.float32)
        ob = q if ob is None else ob + q
    ob = ob + b_ref[:, :]
    for b in range(_B):
        out_ref[0, b] = ob[:, b * _F:(b + 1) * _F]


def kernel(x, psi, quad_weights, weight, bias):
    xt = x.reshape(_B * _C, _NLAT_IN, _NLON_IN).transpose(1, 2, 0)
    psiR = psi.reshape(_K, _NLAT_OUT, _NLAT_IN, _NLON_IN)
    psw = jnp.stack([
        psiR[:, t, _row_start(t):_row_start(t) + _ROWS, :]
        * quad_weights[_row_start(t):_row_start(t) + _ROWS, :][None]
        for t in range(_NLAT_OUT)
    ])
    psw = jnp.pad(psw, ((0, 0), (0, 0), (0, _RPAD - _ROWS), (0, 0)))
    wt = jnp.transpose(weight, (2, 1, 0))
    eyeb = jnp.eye(_B, dtype=jnp.float32)
    wbd = jnp.einsum('kcf,ab->kacbf', wt, eyeb).reshape(_K, _B * _C, _B * _F)
    br = jnp.tile(bias, _B).reshape(1, _B * _F)
    out = pl.pallas_call(
        _disco_kernel,
        grid=(_NLAT_OUT,),
        in_specs=[
            pl.BlockSpec((1, _K, _RPAD, _NLON_IN), lambda t: (t, 0, 0, 0)),
            pl.BlockSpec((_NLAT_IN, _NLON_IN, _B * _C), lambda t: (0, 0, 0)),
            pl.BlockSpec((_K, _B * _C, _B * _F), lambda t: (0, 0, 0)),
            pl.BlockSpec((1, _B * _F), lambda t: (0, 0)),
        ],
        out_specs=pl.BlockSpec((1, _B, _NLON_OUT, _F), lambda t: (t, 0, 0, 0)),
        out_shape=jax.ShapeDtypeStruct((_NLAT_OUT, _B, _NLON_OUT, _F),
                                       jnp.float32),
    )(psw, xt, wbd, br)
    return out.transpose(1, 3, 0, 2)


# speedup vs baseline: 14.1355x; 1.0225x over previous
import jax
import jax.numpy as jnp
from jax.experimental import pallas as pl
from jax.experimental.pallas import tpu as pltpu

_B, _C, _F = 2, 64, 64
_NLAT_IN, _NLON_IN = 64, 128
_NLAT_OUT, _NLON_OUT = 32, 64
_K = 3
_ROWS = 6

_NT = (((1,), (1,)), ((), ()))


def _disco_kernel(psi_ref, qw_ref, x_ref, wbd_ref, b_ref, out_ref):
    t = pl.program_id(0)
    i0 = jnp.clip(2 * t - 2, 0, _NLAT_IN - _ROWS)
    qw = qw_ref[pl.ds(i0, _ROWS), :]
    P = psi_ref[:, 0, pl.ds(i0, _ROWS), :] * qw[None]
    kblocks = []
    for k in range(_K):
        rs = []
        for r in range(_ROWS):
            v = P[k, r, :]
            ct0 = jnp.broadcast_to(v[None, :], (_NLON_OUT, _NLON_IN))
            rs.append(pltpu.roll(ct0, 0, axis=1, stride=2, stride_axis=0))
        kblocks.append(jnp.concatenate(rs, axis=1))
    ct = jnp.concatenate(kblocks, axis=0)
    xw = x_ref[:, pl.ds(i0 * _NLON_IN, _ROWS * _NLON_IN)]
    y = jax.lax.dot_general(ct, xw, _NT,
                            preferred_element_type=jnp.float32)
    ob = None
    for k in range(_K):
        q = jax.lax.dot_general(wbd_ref[k],
                                y[k * _NLON_OUT:(k + 1) * _NLON_OUT, :], _NT,
                                preferred_element_type=jnp.float32)
        ob = q if ob is None else ob + q
    ob = ob + b_ref[:, :]
    out_ref[:, t, :] = ob


def kernel(x, psi, quad_weights, weight, bias):
    xf = x.reshape(_B * _C, _NLAT_IN * _NLON_IN)
    psi4 = psi.reshape(_K, _NLAT_OUT, _NLAT_IN, _NLON_IN)
    eyeb = jnp.eye(_B, dtype=jnp.float32)
    wbdT = jnp.einsum('fck,ab->kafbc', weight, eyeb).reshape(
        _K, _B * _F, _B * _C)
    br = jnp.tile(bias, _B).reshape(_B * _F, 1)
    out = pl.pallas_call(
        _disco_kernel,
        grid=(_NLAT_OUT,),
        in_specs=[
            pl.BlockSpec((_K, 1, _NLAT_IN, _NLON_IN), lambda t: (0, t, 0, 0)),
            pl.BlockSpec((_NLAT_IN, 1), lambda t: (0, 0)),
            pl.BlockSpec((_B * _C, _NLAT_IN * _NLON_IN), lambda t: (0, 0)),
            pl.BlockSpec((_K, _B * _F, _B * _C), lambda t: (0, 0, 0)),
            pl.BlockSpec((_B * _F, 1), lambda t: (0, 0)),
        ],
        out_specs=pl.BlockSpec((_B * _F, _NLAT_OUT, _NLON_OUT),
                               lambda t: (0, 0, 0)),
        out_shape=jax.ShapeDtypeStruct((_B * _F, _NLAT_OUT, _NLON_OUT),
                                       jnp.float32),
    )(psi4, quad_weights, xf, wbdT, br)
    return out.reshape(_B, _F, _NLAT_OUT, _NLON_OUT)


# speedup vs baseline: 19.1748x; 1.3565x over previous
import jax
import jax.numpy as jnp
from jax.experimental import pallas as pl
from jax.experimental.pallas import tpu as pltpu

_B, _C, _F = 2, 64, 64
_NLAT_IN, _NLON_IN = 64, 128
_NLAT_OUT, _NLON_OUT = 32, 64
_K = 3
_ROWS = 6

_NT = (((1,), (1,)), ((), ()))
_TPER = 4


def _disco_kernel(psi_ref, qw_ref, x_ref, wbd_ref, b_ref, out_ref):
    step = pl.program_id(0)
    for tl in range(_TPER):
        t = step * _TPER + tl
        i0 = jnp.clip(2 * t - 2, 0, _NLAT_IN - _ROWS)
        qw = qw_ref[pl.ds(i0, _ROWS), :]
        P = psi_ref[:, tl, pl.ds(i0, _ROWS), :] * qw[None]
        kblocks = []
        for k in range(_K):
            rs = []
            for r in range(_ROWS):
                v = P[k, r, :]
                ct0 = jnp.broadcast_to(v[None, :], (_NLON_OUT, _NLON_IN))
                rs.append(pltpu.roll(ct0, 0, axis=1, stride=2, stride_axis=0))
            kblocks.append(jnp.concatenate(rs, axis=1))
        ct = jnp.concatenate(kblocks, axis=0)
        xw = x_ref[:, pl.ds(i0 * _NLON_IN, _ROWS * _NLON_IN)]
        y = jax.lax.dot_general(ct, xw, _NT,
                                preferred_element_type=jnp.float32)
        ob = None
        for k in range(_K):
            q = jax.lax.dot_general(wbd_ref[k],
                                    y[k * _NLON_OUT:(k + 1) * _NLON_OUT, :],
                                    _NT,
                                    preferred_element_type=jnp.float32)
            ob = q if ob is None else ob + q
        ob = ob + b_ref[:, :]
        out_ref[:, t, :] = ob


def kernel(x, psi, quad_weights, weight, bias):
    xf = x.reshape(_B * _C, _NLAT_IN * _NLON_IN)
    psi4 = psi.reshape(_K, _NLAT_OUT, _NLAT_IN, _NLON_IN)
    eyeb = jnp.eye(_B, dtype=jnp.float32)
    wbdT = jnp.einsum('fck,ab->kafbc', weight, eyeb).reshape(
        _K, _B * _F, _B * _C)
    br = jnp.tile(bias, _B).reshape(_B * _F, 1)
    out = pl.pallas_call(
        _disco_kernel,
        grid=(_NLAT_OUT // _TPER,),
        in_specs=[
            pl.BlockSpec((_K, _TPER, _NLAT_IN, _NLON_IN),
                         lambda s: (0, s, 0, 0)),
            pl.BlockSpec((_NLAT_IN, 1), lambda t: (0, 0)),
            pl.BlockSpec((_B * _C, _NLAT_IN * _NLON_IN), lambda t: (0, 0)),
            pl.BlockSpec((_K, _B * _F, _B * _C), lambda t: (0, 0, 0)),
            pl.BlockSpec((_B * _F, 1), lambda t: (0, 0)),
        ],
        out_specs=pl.BlockSpec((_B * _F, _NLAT_OUT, _NLON_OUT),
                               lambda t: (0, 0, 0)),
        out_shape=jax.ShapeDtypeStruct((_B * _F, _NLAT_OUT, _NLON_OUT),
                                       jnp.float32),
    )(psi4, quad_weights, xf, wbdT, br)
    return out.reshape(_B, _F, _NLAT_OUT, _NLON_OUT)


# speedup vs baseline: 20.3374x; 1.0606x over previous
import jax
import jax.numpy as jnp
from jax.experimental import pallas as pl
from jax.experimental.pallas import tpu as pltpu

_B, _C, _F = 2, 64, 64
_NLAT_IN, _NLON_IN = 64, 128
_NLAT_OUT, _NLON_OUT = 32, 64
_K = 3
_ROWS = 6

_NT = (((1,), (1,)), ((), ()))
_TPER = 8


def _disco_kernel(psi_ref, qw_ref, x_ref, wbd_ref, b_ref, out_ref):
    step = pl.program_id(0)
    for tl in range(_TPER):
        t = step * _TPER + tl
        i0 = jnp.clip(2 * t - 2, 0, _NLAT_IN - _ROWS)
        qw = qw_ref[pl.ds(i0, _ROWS), :]
        P = psi_ref[:, tl, pl.ds(i0, _ROWS), :] * qw[None]
        kblocks = []
        for k in range(_K):
            rs = []
            for r in range(_ROWS):
                v = P[k, r, :]
                ct0 = jnp.broadcast_to(v[None, :], (_NLON_OUT, _NLON_IN))
                rs.append(pltpu.roll(ct0, 0, axis=1, stride=2, stride_axis=0))
            kblocks.append(jnp.concatenate(rs, axis=1))
        ct = jnp.concatenate(kblocks, axis=0)
        xw = x_ref[:, pl.ds(i0 * _NLON_IN, _ROWS * _NLON_IN)]
        y = jax.lax.dot_general(ct, xw, _NT,
                                preferred_element_type=jnp.float32)
        ob = None
        for k in range(_K):
            q = jax.lax.dot_general(wbd_ref[k],
                                    y[k * _NLON_OUT:(k + 1) * _NLON_OUT, :],
                                    _NT,
                                    preferred_element_type=jnp.float32)
            ob = q if ob is None else ob + q
        ob = ob + b_ref[:, :]
        out_ref[:, t, :] = ob


def kernel(x, psi, quad_weights, weight, bias):
    xf = x.reshape(_B * _C, _NLAT_IN * _NLON_IN)
    psi4 = psi.reshape(_K, _NLAT_OUT, _NLAT_IN, _NLON_IN)
    eyeb = jnp.eye(_B, dtype=jnp.float32)
    wbdT = jnp.einsum('fck,ab->kafbc', weight, eyeb).reshape(
        _K, _B * _F, _B * _C)
    br = jnp.tile(bias, _B).reshape(_B * _F, 1)
    out = pl.pallas_call(
        _disco_kernel,
        grid=(_NLAT_OUT // _TPER,),
        in_specs=[
            pl.BlockSpec((_K, _TPER, _NLAT_IN, _NLON_IN),
                         lambda s: (0, s, 0, 0)),
            pl.BlockSpec((_NLAT_IN, 1), lambda t: (0, 0)),
            pl.BlockSpec((_B * _C, _NLAT_IN * _NLON_IN), lambda t: (0, 0)),
            pl.BlockSpec((_K, _B * _F, _B * _C), lambda t: (0, 0, 0)),
            pl.BlockSpec((_B * _F, 1), lambda t: (0, 0)),
        ],
        out_specs=pl.BlockSpec((_B * _F, _NLAT_OUT, _NLON_OUT),
                               lambda t: (0, 0, 0)),
        out_shape=jax.ShapeDtypeStruct((_B * _F, _NLAT_OUT, _NLON_OUT),
                                       jnp.float32),
    )(psi4, quad_weights, xf, wbdT, br)
    return out.reshape(_B, _F, _NLAT_OUT, _NLON_OUT)


# speedup vs baseline: 20.6586x; 1.0158x over previous
import jax
import jax.numpy as jnp
from jax.experimental import pallas as pl
from jax.experimental.pallas import tpu as pltpu

_B, _C, _F = 2, 64, 64
_NLAT_IN, _NLON_IN = 64, 128
_NLAT_OUT, _NLON_OUT = 32, 64
_K = 3
_ROWS = 6

_NT = (((1,), (1,)), ((), ()))
_TPER = 32


def _disco_kernel(psi_ref, qw_ref, x_ref, wbd_ref, b_ref, out_ref):
    step = pl.program_id(0)
    for tl in range(_TPER):
        t = step * _TPER + tl
        i0 = jnp.clip(2 * t - 2, 0, _NLAT_IN - _ROWS)
        qw = qw_ref[pl.ds(i0, _ROWS), :]
        P = psi_ref[:, tl, pl.ds(i0, _ROWS), :] * qw[None]
        kblocks = []
        for k in range(_K):
            rs = []
            for r in range(_ROWS):
                v = P[k, r, :]
                ct0 = jnp.broadcast_to(v[None, :], (_NLON_OUT, _NLON_IN))
                rs.append(pltpu.roll(ct0, 0, axis=1, stride=2, stride_axis=0))
            kblocks.append(jnp.concatenate(rs, axis=1))
        ct = jnp.concatenate(kblocks, axis=0)
        xw = x_ref[:, pl.ds(i0 * _NLON_IN, _ROWS * _NLON_IN)]
        y = jax.lax.dot_general(ct, xw, _NT,
                                preferred_element_type=jnp.float32)
        ob = None
        for k in range(_K):
            q = jax.lax.dot_general(wbd_ref[k],
                                    y[k * _NLON_OUT:(k + 1) * _NLON_OUT, :],
                                    _NT,
                                    preferred_element_type=jnp.float32)
            ob = q if ob is None else ob + q
        ob = ob + b_ref[:, :]
        out_ref[:, t, :] = ob


def kernel(x, psi, quad_weights, weight, bias):
    xf = x.reshape(_B * _C, _NLAT_IN * _NLON_IN)
    psi4 = psi.reshape(_K, _NLAT_OUT, _NLAT_IN, _NLON_IN)
    eyeb = jnp.eye(_B, dtype=jnp.float32)
    wbdT = jnp.einsum('fck,ab->kafbc', weight, eyeb).reshape(
        _K, _B * _F, _B * _C)
    br = jnp.tile(bias, _B).reshape(_B * _F, 1)
    out = pl.pallas_call(
        _disco_kernel,
        grid=(_NLAT_OUT // _TPER,),
        in_specs=[
            pl.BlockSpec((_K, _TPER, _NLAT_IN, _NLON_IN),
                         lambda s: (0, s, 0, 0)),
            pl.BlockSpec((_NLAT_IN, 1), lambda t: (0, 0)),
            pl.BlockSpec((_B * _C, _NLAT_IN * _NLON_IN), lambda t: (0, 0)),
            pl.BlockSpec((_K, _B * _F, _B * _C), lambda t: (0, 0, 0)),
            pl.BlockSpec((_B * _F, 1), lambda t: (0, 0)),
        ],
        out_specs=pl.BlockSpec((_B * _F, _NLAT_OUT, _NLON_OUT),
                               lambda t: (0, 0, 0)),
        out_shape=jax.ShapeDtypeStruct((_B * _F, _NLAT_OUT, _NLON_OUT),
                                       jnp.float32),
    )(psi4, quad_weights, xf, wbdT, br)
    return out.reshape(_B, _F, _NLAT_OUT, _NLON_OUT)
